# baseline (speedup 1.0000x reference)
import functools
import math

import jax
import jax.numpy as jnp
from jax import lax
from jax.experimental import pallas as pl
from jax.experimental.pallas import tpu as pltpu
from jax.experimental.pallas import tpu_sc as plsc

NUM_BUCKETS = 32
NUM_HEADS = 16
MAX_DISTANCE = 128
Q = 2048
K = 2048
GROUP = 8
DW8 = 4224
PW = 3072
NRES = 16
GPR = 8
NGROUPS_H = Q // GROUP


def _table_body(wt_ref, out_ref):
    i = lax.broadcasted_iota(jnp.int32, (GROUP, DW8), 0)
    x = lax.broadcasted_iota(jnp.int32, (GROUP, DW8), 1)
    n = jnp.maximum((Q - GROUP) + i - x, 0)
    max_exact = NUM_BUCKETS // 2
    nf = n.astype(jnp.float32)
    val_if_large = max_exact + (
        jnp.log(nf / max_exact + 1e-09)
        / math.log(MAX_DISTANCE / max_exact)
        * (NUM_BUCKETS - max_exact)
    ).astype(jnp.int32)
    val_if_large = jnp.minimum(val_if_large, NUM_BUCKETS - 1)
    bkt = jnp.where(n < max_exact, n, val_if_large)
    for row in range(GROUP):
        b_iota = lax.broadcasted_iota(jnp.int32, (NUM_BUCKETS, DW8), 0)
        onehot = (bkt[row : row + 1, :] == b_iota).astype(jnp.float32)
        out_ref[:, row, :] = lax.dot_general(
            wt_ref[...],
            onehot,
            (((1,), (0,)), ((), ())),
            precision=lax.Precision.HIGHEST,
            preferred_element_type=jnp.float32,
        )


_build_table = pl.pallas_call(
    _table_body,
    out_shape=jax.ShapeDtypeStruct((NUM_HEADS, GROUP, DW8), jnp.float32),
)


def _tb(ph):
    half, r = divmod(ph, NRES)
    return 1144 - 1024 * half - 8 * r


def _expand_body(ptab_hbm, out_hbm, buf, sem0, sem1):
    c = lax.axis_index("c")
    s = lax.axis_index("s")
    sems = (sem0, sem1)

    def fire_res(r, bi):
        for m in range(GPR):
            q0 = 1024 * c + 8 * r + 128 * m
            boff = 896 - 128 * m
            pltpu.make_async_copy(
                buf.at[bi, :, pl.ds(boff, K)],
                out_hbm.at[0, s, pl.ds(q0, GROUP), :],
                sems[bi],
            ).start()

    def drain_res(bi):
        for _ in range(GPR):
            pltpu.make_async_copy(
                buf.at[bi, :, pl.ds(0, K)],
                out_hbm.at[0, 0, pl.ds(0, GROUP), :],
                sems[bi],
            ).wait()

    for r in range(NRES):
        bi = r % 2
        if r >= 2:
            drain_res(bi)
        pltpu.sync_copy(ptab_hbm.at[s, c * NRES + r], buf.at[bi])
        fire_res(r, bi)
    drain_res(0)
    drain_res(1)


@functools.cache
def _expand():
    return pl.kernel(
        _expand_body,
        out_type=jax.ShapeDtypeStruct((1, NUM_HEADS, Q, K), jnp.float32),
        mesh=plsc.VectorSubcoreMesh(core_axis_name="c", subcore_axis_name="s"),
        scratch_types=[
            pltpu.VMEM((2, GROUP, PW), jnp.float32),
            pltpu.SemaphoreType.DMA,
            pltpu.SemaphoreType.DMA,
        ],
    )


def kernel(qlen, klen, W):
    wt = W.T
    dsh = _build_table(wt)
    ptab = jnp.stack(
        [lax.slice_in_dim(dsh, _tb(ph), _tb(ph) + PW, axis=2) for ph in range(32)],
        axis=1,
    )
    return _expand()(ptab)


# speedup vs baseline: 1.7451x; 1.7451x over previous
import functools
import math

import jax
import jax.numpy as jnp
from jax import lax
from jax.experimental import pallas as pl
from jax.experimental.pallas import tpu as pltpu
from jax.experimental.pallas import tpu_sc as plsc

NUM_BUCKETS = 32
NUM_HEADS = 16
MAX_DISTANCE = 128
Q = 2048
K = 2048
GROUP = 8
DW8 = 4224
PW = 3072
NRES = 16
GPR = 8
CHUNKS = PW // 16


def _table_body(wt_ref, out_ref):
    for p in range(2):
        i = lax.broadcasted_iota(jnp.int32, (GROUP, DW8), 0)
        x = lax.broadcasted_iota(jnp.int32, (GROUP, DW8), 1)
        n = jnp.maximum((Q - GROUP) - 8 * p + i - x, 0)
        max_exact = NUM_BUCKETS // 2
        nf = n.astype(jnp.float32)
        val_if_large = max_exact + (
            jnp.log(nf / max_exact + 1e-09)
            / math.log(MAX_DISTANCE / max_exact)
            * (NUM_BUCKETS - max_exact)
        ).astype(jnp.int32)
        val_if_large = jnp.minimum(val_if_large, NUM_BUCKETS - 1)
        bkt = jnp.where(n < max_exact, n, val_if_large)
        for row in range(GROUP):
            b_iota = lax.broadcasted_iota(jnp.int32, (NUM_BUCKETS, DW8), 0)
            onehot = (bkt[row : row + 1, :] == b_iota).astype(jnp.float32)
            out_ref[p, :, row, :] = lax.dot_general(
                wt_ref[...],
                onehot,
                (((1,), (0,)), ((), ())),
                precision=lax.Precision.HIGHEST,
                preferred_element_type=jnp.float32,
            )


_build_table = pl.pallas_call(
    _table_body,
    out_shape=jax.ShapeDtypeStruct((2, NUM_HEADS, GROUP, DW8), jnp.float32),
)


def _expand_body(dsh_hbm, out_hbm, dsh_v, buf, sem0, sem1):
    c = lax.axis_index("c")
    s = lax.axis_index("s")
    sems = (sem0, sem1)

    pltpu.sync_copy(dsh_hbm.at[0, s], dsh_v.at[0])
    pltpu.sync_copy(dsh_hbm.at[1, s], dsh_v.at[1])

    def build_res(r, bi):
        pr = (r + 1) % 2
        off = 1144 - 8 * r - 8 * pr - 1024 * c

        def chunk(j, carry):
            y0 = j * 16
            for i in range(GROUP):
                buf[bi, i, pl.ds(y0, 16)] = dsh_v[pr, i, pl.ds(off + y0, 16)]
            return carry

        lax.fori_loop(0, CHUNKS, chunk, 0)

    def fire_res(r, bi):
        for m in range(GPR):
            q0 = 1024 * c + 8 * r + 128 * m
            boff = 896 - 128 * m
            pltpu.make_async_copy(
                buf.at[bi, :, pl.ds(boff, K)],
                out_hbm.at[0, s, pl.ds(q0, GROUP), :],
                sems[bi],
            ).start()

    def drain_res(bi):
        for _ in range(GPR):
            pltpu.make_async_copy(
                buf.at[bi, :, pl.ds(0, K)],
                out_hbm.at[0, 0, pl.ds(0, GROUP), :],
                sems[bi],
            ).wait()

    for r in range(NRES):
        bi = r % 2
        if r >= 2:
            drain_res(bi)
        build_res(r, bi)
        fire_res(r, bi)
    drain_res(0)
    drain_res(1)


@functools.cache
def _expand():
    return pl.kernel(
        _expand_body,
        out_type=jax.ShapeDtypeStruct((1, NUM_HEADS, Q, K), jnp.float32),
        mesh=plsc.VectorSubcoreMesh(core_axis_name="c", subcore_axis_name="s"),
        scratch_types=[
            pltpu.VMEM((2, GROUP, DW8), jnp.float32),
            pltpu.VMEM((2, GROUP, PW), jnp.float32),
            pltpu.SemaphoreType.DMA,
            pltpu.SemaphoreType.DMA,
        ],
    )


def kernel(qlen, klen, W):
    wt = W.T
    dsh = _build_table(wt)
    return _expand()(dsh)
